# baseline (speedup 1.0000x reference)
import functools

import jax
import jax.numpy as jnp
from jax import lax
from jax.experimental import pallas as pl
from jax.experimental.pallas import tpu as pltpu
from jax.experimental.pallas import tpu_sc as plsc

BATCH = 4
SEQ = 2048
EMB_DIM = 128
NUM_CORES = 2
NUM_SUBCORES = 16
NUM_WORKERS = NUM_CORES * NUM_SUBCORES
WIN = SEQ // NUM_WORKERS
LANES = 16
COL_CHUNKS = EMB_DIM // LANES


def _emb_body(x_hbm, tok_hbm, pos_hbm, out_hbm, idx_v, rows_v, pbuf, semi, semp, semg, semo):
    wid = lax.axis_index("s") * NUM_CORES + lax.axis_index("c")
    p0 = wid * WIN

    i_cps = [
        pltpu.async_copy(x_hbm.at[b, pl.ds(p0, WIN)], idx_v.at[b], semi.at[b])
        for b in range(BATCH)
    ]
    pos_cp = pltpu.async_copy(pos_hbm.at[pl.ds(p0, WIN)], pbuf, semp.at[0])
    pos_cp.wait()

    g_cps = []
    for b in range(BATCH):
        def repl_row(r, carry, _b=b):
            for c in range(COL_CHUNKS):
                sl = pl.ds(c * LANES, LANES)
                rows_v[_b * WIN + r, sl] = pbuf[r, sl]
            return carry

        lax.fori_loop(0, WIN, repl_row, 0, unroll=4)
        i_cps[b].wait()
        g_cps.append(
            pltpu.async_copy(
                tok_hbm.at[idx_v.at[b]],
                rows_v.at[pl.ds(b * WIN, WIN)],
                semg.at[b],
                add=True,
            )
        )
    o_cps = []
    for b in range(BATCH):
        g_cps[b].wait()
        o_cps.append(
            pltpu.async_copy(
                rows_v.at[pl.ds(b * WIN, WIN)],
                out_hbm.at[b, pl.ds(p0, WIN)],
                semo.at[b],
            )
        )
    for cp in o_cps:
        cp.wait()


_emb_call = functools.partial(
    pl.kernel,
    out_type=jax.ShapeDtypeStruct((BATCH, SEQ, EMB_DIM), jnp.float32),
    mesh=plsc.VectorSubcoreMesh(core_axis_name="c", subcore_axis_name="s"),
    scratch_types=[
        pltpu.VMEM((BATCH, WIN), jnp.int32),
        pltpu.VMEM((BATCH * WIN, EMB_DIM), jnp.float32),
        pltpu.VMEM((WIN, EMB_DIM), jnp.float32),
        pltpu.SemaphoreType.DMA((BATCH,)),
        pltpu.SemaphoreType.DMA((1,)),
        pltpu.SemaphoreType.DMA((BATCH,)),
        pltpu.SemaphoreType.DMA((BATCH,)),
    ],
)(_emb_body)


def kernel(x, tok_emb, pos_emb):
    return _emb_call(x.astype(jnp.int32), tok_emb, pos_emb)


# speedup vs baseline: 1.1145x; 1.1145x over previous
import functools

import jax
import jax.numpy as jnp
from jax import lax
from jax.experimental import pallas as pl
from jax.experimental.pallas import tpu as pltpu
from jax.experimental.pallas import tpu_sc as plsc

BATCH = 4
SEQ = 2048
EMB_DIM = 128
NUM_CORES = 2
NUM_SUBCORES = 16
NUM_WORKERS = NUM_CORES * NUM_SUBCORES
ROWS_PER_WORKER = BATCH * SEQ // NUM_WORKERS
WORKERS_PER_BATCH = SEQ // ROWS_PER_WORKER
NCHUNK = 4
CHUNK = ROWS_PER_WORKER // NCHUNK


def _emb_body(x_hbm, tok_hbm, pos_hbm, out_hbm, idx_v, rows_v, semi, semp, semg, semo):
    wid = lax.axis_index("s") * NUM_CORES + lax.axis_index("c")
    b = lax.div(wid, WORKERS_PER_BATCH)
    s0 = lax.rem(wid, WORKERS_PER_BATCH) * ROWS_PER_WORKER

    idx_cp = pltpu.async_copy(
        x_hbm.at[b, pl.ds(s0, ROWS_PER_WORKER)], idx_v, semi.at[0]
    )
    pos_cps = [
        pltpu.async_copy(
            pos_hbm.at[pl.ds(s0 + c * CHUNK, CHUNK)],
            rows_v.at[pl.ds(c * CHUNK, CHUNK)],
            semp.at[c],
        )
        for c in range(NCHUNK)
    ]
    idx_cp.wait()
    g_cps = []
    for c in range(NCHUNK):
        pos_cps[c].wait()
        g_cps.append(
            pltpu.async_copy(
                tok_hbm.at[idx_v.at[pl.ds(c * CHUNK, CHUNK)]],
                rows_v.at[pl.ds(c * CHUNK, CHUNK)],
                semg.at[c],
                add=True,
            )
        )
    o_cps = []
    for c in range(NCHUNK):
        g_cps[c].wait()
        o_cps.append(
            pltpu.async_copy(
                rows_v.at[pl.ds(c * CHUNK, CHUNK)],
                out_hbm.at[b, pl.ds(s0 + c * CHUNK, CHUNK)],
                semo.at[c],
            )
        )
    for cp in o_cps:
        cp.wait()


_emb_call = functools.partial(
    pl.kernel,
    out_type=jax.ShapeDtypeStruct((BATCH, SEQ, EMB_DIM), jnp.float32),
    mesh=plsc.VectorSubcoreMesh(core_axis_name="c", subcore_axis_name="s"),
    scratch_types=[
        pltpu.VMEM((ROWS_PER_WORKER,), jnp.int32),
        pltpu.VMEM((ROWS_PER_WORKER, EMB_DIM), jnp.float32),
        pltpu.SemaphoreType.DMA((1,)),
        pltpu.SemaphoreType.DMA((NCHUNK,)),
        pltpu.SemaphoreType.DMA((NCHUNK,)),
        pltpu.SemaphoreType.DMA((NCHUNK,)),
    ],
)(_emb_body)


def kernel(x, tok_emb, pos_emb):
    return _emb_call(x.astype(jnp.int32), tok_emb, pos_emb)
